# baseline (speedup 1.0000x reference)
import functools

import jax
import jax.numpy as jnp
from jax import lax
from jax.experimental import pallas as pl
from jax.experimental.pallas import tpu as pltpu
from jax.experimental.pallas import tpu_sc as plsc

_NC = 2
_NS = 16
_CH = 125
_K = 2


def _agg_build(N, H, E):
    HH = H // 2
    assert E % (_NS * _CH) == 0, (N, H, E)
    NCH = E // (_NS * _CH)
    assert NCH % (2 * _K) == 0
    rows_pt = (N // _NS) // 8 * 8
    tail = N - rows_pt * _NS
    assert tail % 8 == 0
    mesh = plsc.VectorSubcoreMesh(
        core_axis_name="c", subcore_axis_name="s",
        num_cores=_NC, num_subcores=_NS)

    @functools.partial(
        pl.kernel,
        out_type=jax.ShapeDtypeStruct((_NC, N, HH), jnp.float32),
        mesh=mesh,
        compiler_params=pltpu.CompilerParams(use_tc_tiling_on_sc=False),
        scratch_types=[
            pltpu.VMEM((NCH, _CH), jnp.int32),
            pltpu.VMEM((NCH, _CH), jnp.int32),
            pltpu.VMEM((2 * _K, _CH, HH), jnp.float32),
            pltpu.VMEM_SHARED((N, HH), jnp.float32),
            pltpu.SemaphoreType.DMA,
            pltpu.SemaphoreType.DMA,
        ],
    )
    def agg(xs_hbm, src_hbm, dst_hbm, out_hbm,
            srcv, dstv, gbuf, accum, gsem, ssem):
        cid = lax.axis_index("c")
        sid = lax.axis_index("s")
        xh = xs_hbm.at[cid]
        pltpu.sync_copy(src_hbm.at[sid], srcv)
        pltpu.sync_copy(dst_hbm.at[sid], dstv)
        r0 = sid * rows_pt
        pltpu.sync_copy(xh.at[pl.ds(r0, rows_pt)],
                        accum.at[pl.ds(r0, rows_pt)])
        if tail:
            @pl.when(sid == 0)
            def _tail_init():
                pltpu.sync_copy(xh.at[pl.ds(N - tail, tail)],
                                accum.at[pl.ds(N - tail, tail)])

        plsc.subcore_barrier()

        NG = NCH // _K
        for j in range(_K):
            pltpu.async_copy(xh.at[srcv.at[j]], gbuf.at[j], gsem)

        def gen(g, carry):
            p = lax.rem(g, 2)
            c0 = g * _K

            @pl.when(g >= 1)
            def _drain_prev_scatters():
                for j in range(_K):
                    pltpu.make_async_copy(
                        gbuf.at[(1 - p) * _K + j],
                        accum.at[dstv.at[c0 - _K + j]], ssem).wait()

            for j in range(_K):
                pltpu.make_async_copy(
                    xh.at[srcv.at[c0 + j]], gbuf.at[p * _K + j],
                    gsem).wait()
            for j in range(_K):
                pltpu.async_copy(
                    gbuf.at[p * _K + j], accum.at[dstv.at[c0 + j]], ssem,
                    add=True)

            @pl.when(g + 1 < NG)
            def _fire_next_gathers():
                for j in range(_K):
                    pltpu.async_copy(
                        xh.at[srcv.at[c0 + _K + j]],
                        gbuf.at[(1 - p) * _K + j], gsem)

            return carry

        lax.fori_loop(0, NG, gen, 0)
        pfin = lax.rem(NG - 1, 2)
        for j in range(_K):
            pltpu.make_async_copy(
                gbuf.at[pfin * _K + j],
                accum.at[dstv.at[(NG - 1) * _K + j]], ssem).wait()
        plsc.subcore_barrier()
        pltpu.sync_copy(accum.at[pl.ds(r0, rows_pt)],
                        out_hbm.at[cid, pl.ds(r0, rows_pt)])
        if tail:
            @pl.when(sid == 0)
            def _tail_out():
                pltpu.sync_copy(accum.at[pl.ds(N - tail, tail)],
                                out_hbm.at[cid, pl.ds(N - tail, tail)])

    return agg


_BR = 2000


def _mlp_fused(aggs, w1a, w1b, b1, gamma, beta, w2, b2, split):
    _, N, HH = aggs.shape
    H = w1a.shape[1]
    NB = N // _BR
    inv_n = 1.0 / N

    def body(a_ref, w1a_ref, w1b_ref, b1_ref, g_ref, be_ref, w2_ref,
             b2_ref, o_ref, h_scr, st_scr):
        i = pl.program_id(0)
        j = pl.program_id(1)

        @pl.when(i == 0)
        def _phase0():
            h = jnp.dot(a_ref[0], w1a_ref[...],
                        preferred_element_type=jnp.float32)
            h += jnp.dot(a_ref[1], w1b_ref[...],
                         preferred_element_type=jnp.float32)
            h = h + b1_ref[...]
            h_scr[pl.ds(j * _BR, _BR), :] = h

            @pl.when(j == 0)
            def _init():
                st_scr[...] = jnp.zeros_like(st_scr)

            st_scr[0:1, :] += jnp.sum(h, axis=0, keepdims=True)
            st_scr[1:2, :] += jnp.sum(h * h, axis=0, keepdims=True)

        @pl.when(i == 1)
        def _phase1():
            st = st_scr[...]
            mu = st[0:1, :] * inv_n
            var = st[1:2, :] * inv_n - mu * mu
            hn = (h_scr[pl.ds(j * _BR, _BR), :] - mu) * lax.rsqrt(var + 1e-5)
            hn = hn * g_ref[...] + be_ref[...]
            hn = jnp.maximum(hn, 0.0)
            y = jnp.dot(hn, w2_ref[...], preferred_element_type=jnp.float32)
            y = y + b2_ref[...]
            y = jnp.where(y > 0, y, 0.1 * (jnp.exp(y) - 1.0))
            if split:
                o_ref[0] = y[:, :HH]
                o_ref[1] = y[:, HH:]
            else:
                o_ref[...] = y

    if split:
        out_spec = pl.BlockSpec(
            (2, _BR, HH), lambda i, j: (0, jnp.where(i == 1, j, 0), 0))
        out_shape = jax.ShapeDtypeStruct((2, N, HH), jnp.float32)
    else:
        out_spec = pl.BlockSpec(
            (_BR, H), lambda i, j: (jnp.where(i == 1, j, 0), 0))
        out_shape = jax.ShapeDtypeStruct((N, H), jnp.float32)

    return pl.pallas_call(
        body,
        grid=(2, NB),
        in_specs=[
            pl.BlockSpec((2, _BR, HH),
                         lambda i, j: (0, jnp.where(i == 0, j, NB - 1), 0)),
            pl.BlockSpec((HH, H), lambda i, j: (0, 0)),
            pl.BlockSpec((HH, H), lambda i, j: (0, 0)),
            pl.BlockSpec((1, H), lambda i, j: (0, 0)),
            pl.BlockSpec((1, H), lambda i, j: (0, 0)),
            pl.BlockSpec((1, H), lambda i, j: (0, 0)),
            pl.BlockSpec((H, H), lambda i, j: (0, 0)),
            pl.BlockSpec((1, H), lambda i, j: (0, 0)),
        ],
        out_specs=out_spec,
        out_shape=out_shape,
        scratch_shapes=[
            pltpu.VMEM((N, H), jnp.float32),
            pltpu.VMEM((8, H), jnp.float32),
        ],
    )(aggs, w1a, w1b, b1.reshape(1, H), gamma.reshape(1, H),
      beta.reshape(1, H), w2, b2.reshape(1, H))


def _mlp_pool_clf(aggs, w1a, w1b, b1, gamma, beta, w2, b2,
                  batch, g0, c1a, c1b, c1c, cb1, cw2, cb2):
    _, N, HH = aggs.shape
    H = w1a.shape[1]
    G, NGF = g0.shape
    NCLS = cw2.shape[1]
    NB = N // _BR
    inv_n = 1.0 / N
    b_col = batch.reshape(NB, _BR, 1)
    b_row = batch.reshape(NB, 1, _BR)

    def body(a_ref, w1a_ref, w1b_ref, b1_ref, g_ref, be_ref, w2_ref,
             b2_ref, bc_ref, br_ref, g0_ref, c1a_ref, c1b_ref, c1c_ref,
             cb1_ref, cw2_ref, cb2_ref, o_ref, h_scr, st_scr, sum_scr,
             max_scr, cnt_scr):
        i = pl.program_id(0)
        j = pl.program_id(1)

        @pl.when(i == 0)
        def _phase0():
            h = jnp.dot(a_ref[0], w1a_ref[...],
                        preferred_element_type=jnp.float32)
            h += jnp.dot(a_ref[1], w1b_ref[...],
                         preferred_element_type=jnp.float32)
            h = h + b1_ref[...]
            h_scr[pl.ds(j * _BR, _BR), :] = h

            @pl.when(j == 0)
            def _init():
                st_scr[...] = jnp.zeros_like(st_scr)
                sum_scr[...] = jnp.zeros_like(sum_scr)
                cnt_scr[...] = jnp.zeros_like(cnt_scr)
                max_scr[...] = jnp.full_like(max_scr, -1e30)

            st_scr[0:1, :] += jnp.sum(h, axis=0, keepdims=True)
            st_scr[1:2, :] += jnp.sum(h * h, axis=0, keepdims=True)

        @pl.when(i == 1)
        def _phase1():
            st = st_scr[...]
            mu = st[0:1, :] * inv_n
            var = st[1:2, :] * inv_n - mu * mu
            hn = (h_scr[pl.ds(j * _BR, _BR), :] - mu) * lax.rsqrt(var + 1e-5)
            hn = hn * g_ref[...] + be_ref[...]
            hn = jnp.maximum(hn, 0.0)
            y = jnp.dot(hn, w2_ref[...], preferred_element_type=jnp.float32)
            y = y + b2_ref[...]
            y = jnp.where(y > 0, y, 0.1 * (jnp.exp(y) - 1.0))
            bc = bc_ref[0]
            br = br_ref[0]
            ohT = (lax.broadcasted_iota(jnp.int32, (G, 1), 0) == br
                   ).astype(jnp.float32)
            sum_scr[...] += jnp.dot(ohT, y,
                                    preferred_element_type=jnp.float32)
            cnt_scr[...] += jnp.sum(ohT, axis=1, keepdims=True)
            for g in range(G):
                mg = jnp.where(bc == g, y, -1e30)
                max_scr[g:g + 1, :] = jnp.maximum(
                    max_scr[g:g + 1, :], jnp.max(mg, axis=0, keepdims=True))

            @pl.when(j == NB - 1)
            def _final():
                cnt = cnt_scr[...]
                x1 = sum_scr[...] / jnp.maximum(cnt, 1.0)
                x2 = jnp.where(cnt > 0.0, max_scr[...], 0.0)
                z = (jnp.dot(x1, c1a_ref[...],
                             preferred_element_type=jnp.float32)
                     + jnp.dot(x2, c1b_ref[...],
                               preferred_element_type=jnp.float32)
                     + jnp.dot(g0_ref[...], c1c_ref[...],
                               preferred_element_type=jnp.float32)
                     + cb1_ref[...])
                z = jnp.where(z > 0, z, 0.1 * (jnp.exp(z) - 1.0))
                lg = jnp.dot(z, cw2_ref[...],
                             preferred_element_type=jnp.float32)
                lg = lg + cb2_ref[...]
                m = jnp.max(lg, axis=1, keepdims=True)
                e = jnp.exp(lg - m)
                o_ref[...] = e / jnp.sum(e, axis=1, keepdims=True)

    return pl.pallas_call(
        body,
        grid=(2, NB),
        in_specs=[
            pl.BlockSpec((2, _BR, HH),
                         lambda i, j: (0, jnp.where(i == 0, j, NB - 1), 0)),
            pl.BlockSpec((HH, H), lambda i, j: (0, 0)),
            pl.BlockSpec((HH, H), lambda i, j: (0, 0)),
            pl.BlockSpec((1, H), lambda i, j: (0, 0)),
            pl.BlockSpec((1, H), lambda i, j: (0, 0)),
            pl.BlockSpec((1, H), lambda i, j: (0, 0)),
            pl.BlockSpec((H, H), lambda i, j: (0, 0)),
            pl.BlockSpec((1, H), lambda i, j: (0, 0)),
            pl.BlockSpec((1, _BR, 1),
                         lambda i, j: (jnp.where(i == 1, j, 0), 0, 0)),
            pl.BlockSpec((1, 1, _BR),
                         lambda i, j: (jnp.where(i == 1, j, 0), 0, 0)),
            pl.BlockSpec((G, NGF), lambda i, j: (0, 0)),
            pl.BlockSpec((H, H), lambda i, j: (0, 0)),
            pl.BlockSpec((H, H), lambda i, j: (0, 0)),
            pl.BlockSpec((NGF, H), lambda i, j: (0, 0)),
            pl.BlockSpec((1, H), lambda i, j: (0, 0)),
            pl.BlockSpec((H, NCLS), lambda i, j: (0, 0)),
            pl.BlockSpec((1, NCLS), lambda i, j: (0, 0)),
        ],
        out_specs=pl.BlockSpec((G, NCLS), lambda i, j: (0, 0)),
        out_shape=jax.ShapeDtypeStruct((G, NCLS), jnp.float32),
        scratch_shapes=[
            pltpu.VMEM((N, H), jnp.float32),
            pltpu.VMEM((8, H), jnp.float32),
            pltpu.VMEM((G, H), jnp.float32),
            pltpu.VMEM((G, H), jnp.float32),
            pltpu.VMEM((G, 1), jnp.float32),
        ],
    )(aggs, w1a, w1b, b1.reshape(1, H), gamma.reshape(1, H),
      beta.reshape(1, H), w2, b2.reshape(1, H), b_col, b_row, g0,
      c1a, c1b, c1c, cb1.reshape(1, H), cw2, cb2.reshape(1, NCLS))


def kernel(h0, coord0, g0, edge_index, batch,
           gin0_W1, gin0_b1, gin0_gamma, gin0_beta, gin0_W2, gin0_b2,
           gin1_W1, gin1_b1, gin1_gamma, gin1_beta, gin1_W2, gin1_b2,
           gin2_W1, gin2_b1, gin2_gamma, gin2_beta, gin2_W2, gin2_b2,
           clf_W1, clf_b1, clf_W2, clf_b2):
    x = jnp.concatenate([h0, coord0], axis=1)
    N, H = x.shape
    HH = H // 2
    E = edge_index.shape[1]
    src3 = edge_index[0].reshape(_NS, E // (_NS * _CH), _CH)
    dst3 = edge_index[1].reshape(_NS, E // (_NS * _CH), _CH)
    xs = jnp.stack([x[:, :HH], x[:, HH:]])
    agg_fn = _agg_build(N, H, E)
    params = [
        (gin0_W1, gin0_b1, gin0_gamma, gin0_beta, gin0_W2, gin0_b2),
        (gin1_W1, gin1_b1, gin1_gamma, gin1_beta, gin1_W2, gin1_b2),
        (gin2_W1, gin2_b1, gin2_gamma, gin2_beta, gin2_W2, gin2_b2),
    ]
    w1a = clf_W1[:H]
    w1b = clf_W1[H:2 * H]
    w1c = clf_W1[2 * H:]
    for li, (w1, b1, gamma, beta, w2, b2) in enumerate(params):
        aggs = agg_fn(xs, src3, dst3)
        if li < len(params) - 1:
            xs = _mlp_fused(aggs, w1[:HH], w1[HH:], b1, gamma, beta, w2, b2,
                            split=True)
        else:
            return _mlp_pool_clf(aggs, w1[:HH], w1[HH:], b1, gamma, beta,
                                 w2, b2, batch, g0, w1a, w1b, w1c, clf_b1,
                                 clf_W2, clf_b2)


# speedup vs baseline: 1.0790x; 1.0790x over previous
import functools

import jax
import jax.numpy as jnp
from jax import lax
from jax.experimental import pallas as pl
from jax.experimental.pallas import tpu as pltpu
from jax.experimental.pallas import tpu_sc as plsc

_NC = 2
_NS = 16
_CH = 125
_NB = 4


def _agg_build(N, H, E):
    HH = H // 2
    assert E % (_NS * _CH) == 0, (N, H, E)
    NCH = E // (_NS * _CH)
    assert NCH % _NB == 0
    rows_pt = (N // _NS) // 8 * 8
    tail = N - rows_pt * _NS
    assert tail % 8 == 0
    mesh = plsc.VectorSubcoreMesh(
        core_axis_name="c", subcore_axis_name="s",
        num_cores=_NC, num_subcores=_NS)

    @functools.partial(
        pl.kernel,
        out_type=jax.ShapeDtypeStruct((_NC, N, HH), jnp.float32),
        mesh=mesh,
        compiler_params=pltpu.CompilerParams(use_tc_tiling_on_sc=False),
        scratch_types=[
            pltpu.VMEM((NCH, _CH), jnp.int32),
            pltpu.VMEM((NCH, _CH), jnp.int32),
            pltpu.VMEM((_NB, _CH, HH), jnp.float32),
            pltpu.VMEM_SHARED((N, HH), jnp.float32),
            [pltpu.SemaphoreType.DMA] * _NB,
            [pltpu.SemaphoreType.DMA] * _NB,
        ],
    )
    def agg(xs_hbm, src_hbm, dst_hbm, out_hbm,
            srcv, dstv, gbuf, accum, gsems, ssems):
        cid = lax.axis_index("c")
        sid = lax.axis_index("s")
        xh = xs_hbm.at[cid]
        pltpu.sync_copy(src_hbm.at[sid], srcv)
        pltpu.sync_copy(dst_hbm.at[sid], dstv)
        r0 = sid * rows_pt
        pltpu.sync_copy(xh.at[pl.ds(r0, rows_pt)],
                        accum.at[pl.ds(r0, rows_pt)])
        if tail:
            @pl.when(sid == 0)
            def _tail_init():
                pltpu.sync_copy(xh.at[pl.ds(N - tail, tail)],
                                accum.at[pl.ds(N - tail, tail)])

        plsc.subcore_barrier()

        NR = NCH // _NB
        for b in range(_NB):
            pltpu.async_copy(xh.at[srcv.at[b]], gbuf.at[b], gsems[b])

        def rnd(r, carry):
            c0 = r * _NB
            for b in range(_NB):
                pltpu.make_async_copy(
                    xh.at[srcv.at[c0 + b]], gbuf.at[b], gsems[b]).wait()
                pltpu.async_copy(
                    gbuf.at[b], accum.at[dstv.at[c0 + b]], ssems[b],
                    add=True)

            @pl.when(r + 1 < NR)
            def _refill():
                for b in range(_NB):
                    pltpu.make_async_copy(
                        gbuf.at[b], accum.at[dstv.at[c0 + b]],
                        ssems[b]).wait()
                    pltpu.async_copy(
                        xh.at[srcv.at[c0 + _NB + b]], gbuf.at[b], gsems[b])

            return carry

        lax.fori_loop(0, NR, rnd, 0)
        for b in range(_NB):
            pltpu.make_async_copy(
                gbuf.at[b], accum.at[dstv.at[(NR - 1) * _NB + b]],
                ssems[b]).wait()
        plsc.subcore_barrier()
        pltpu.sync_copy(accum.at[pl.ds(r0, rows_pt)],
                        out_hbm.at[cid, pl.ds(r0, rows_pt)])
        if tail:
            @pl.when(sid == 0)
            def _tail_out():
                pltpu.sync_copy(accum.at[pl.ds(N - tail, tail)],
                                out_hbm.at[cid, pl.ds(N - tail, tail)])

    return agg


_BR = 2000


def _mlp_fused(aggs, w1a, w1b, b1, gamma, beta, w2, b2, split):
    _, N, HH = aggs.shape
    H = w1a.shape[1]
    NB = N // _BR
    inv_n = 1.0 / N

    def body(a_ref, w1a_ref, w1b_ref, b1_ref, g_ref, be_ref, w2_ref,
             b2_ref, o_ref, h_scr, st_scr):
        i = pl.program_id(0)
        j = pl.program_id(1)

        @pl.when(i == 0)
        def _phase0():
            h = jnp.dot(a_ref[0], w1a_ref[...],
                        preferred_element_type=jnp.float32)
            h += jnp.dot(a_ref[1], w1b_ref[...],
                         preferred_element_type=jnp.float32)
            h = h + b1_ref[...]
            h_scr[pl.ds(j * _BR, _BR), :] = h

            @pl.when(j == 0)
            def _init():
                st_scr[...] = jnp.zeros_like(st_scr)

            st_scr[0:1, :] += jnp.sum(h, axis=0, keepdims=True)
            st_scr[1:2, :] += jnp.sum(h * h, axis=0, keepdims=True)

        @pl.when(i == 1)
        def _phase1():
            st = st_scr[...]
            mu = st[0:1, :] * inv_n
            var = st[1:2, :] * inv_n - mu * mu
            hn = (h_scr[pl.ds(j * _BR, _BR), :] - mu) * lax.rsqrt(var + 1e-5)
            hn = hn * g_ref[...] + be_ref[...]
            hn = jnp.maximum(hn, 0.0)
            y = jnp.dot(hn, w2_ref[...], preferred_element_type=jnp.float32)
            y = y + b2_ref[...]
            y = jnp.where(y > 0, y, 0.1 * (jnp.exp(y) - 1.0))
            if split:
                o_ref[0] = y[:, :HH]
                o_ref[1] = y[:, HH:]
            else:
                o_ref[...] = y

    if split:
        out_spec = pl.BlockSpec(
            (2, _BR, HH), lambda i, j: (0, jnp.where(i == 1, j, 0), 0))
        out_shape = jax.ShapeDtypeStruct((2, N, HH), jnp.float32)
    else:
        out_spec = pl.BlockSpec(
            (_BR, H), lambda i, j: (jnp.where(i == 1, j, 0), 0))
        out_shape = jax.ShapeDtypeStruct((N, H), jnp.float32)

    return pl.pallas_call(
        body,
        grid=(2, NB),
        in_specs=[
            pl.BlockSpec((2, _BR, HH),
                         lambda i, j: (0, jnp.where(i == 0, j, NB - 1), 0)),
            pl.BlockSpec((HH, H), lambda i, j: (0, 0)),
            pl.BlockSpec((HH, H), lambda i, j: (0, 0)),
            pl.BlockSpec((1, H), lambda i, j: (0, 0)),
            pl.BlockSpec((1, H), lambda i, j: (0, 0)),
            pl.BlockSpec((1, H), lambda i, j: (0, 0)),
            pl.BlockSpec((H, H), lambda i, j: (0, 0)),
            pl.BlockSpec((1, H), lambda i, j: (0, 0)),
        ],
        out_specs=out_spec,
        out_shape=out_shape,
        scratch_shapes=[
            pltpu.VMEM((N, H), jnp.float32),
            pltpu.VMEM((8, H), jnp.float32),
        ],
    )(aggs, w1a, w1b, b1.reshape(1, H), gamma.reshape(1, H),
      beta.reshape(1, H), w2, b2.reshape(1, H))


def _mlp_pool_clf(aggs, w1a, w1b, b1, gamma, beta, w2, b2,
                  batch, g0, c1a, c1b, c1c, cb1, cw2, cb2):
    _, N, HH = aggs.shape
    H = w1a.shape[1]
    G, NGF = g0.shape
    NCLS = cw2.shape[1]
    NB = N // _BR
    inv_n = 1.0 / N
    b_col = batch.reshape(NB, _BR, 1)
    b_row = batch.reshape(NB, 1, _BR)

    def body(a_ref, w1a_ref, w1b_ref, b1_ref, g_ref, be_ref, w2_ref,
             b2_ref, bc_ref, br_ref, g0_ref, c1a_ref, c1b_ref, c1c_ref,
             cb1_ref, cw2_ref, cb2_ref, o_ref, h_scr, st_scr, sum_scr,
             max_scr, cnt_scr):
        i = pl.program_id(0)
        j = pl.program_id(1)

        @pl.when(i == 0)
        def _phase0():
            h = jnp.dot(a_ref[0], w1a_ref[...],
                        preferred_element_type=jnp.float32)
            h += jnp.dot(a_ref[1], w1b_ref[...],
                         preferred_element_type=jnp.float32)
            h = h + b1_ref[...]
            h_scr[pl.ds(j * _BR, _BR), :] = h

            @pl.when(j == 0)
            def _init():
                st_scr[...] = jnp.zeros_like(st_scr)
                sum_scr[...] = jnp.zeros_like(sum_scr)
                cnt_scr[...] = jnp.zeros_like(cnt_scr)
                max_scr[...] = jnp.full_like(max_scr, -1e30)

            st_scr[0:1, :] += jnp.sum(h, axis=0, keepdims=True)
            st_scr[1:2, :] += jnp.sum(h * h, axis=0, keepdims=True)

        @pl.when(i == 1)
        def _phase1():
            st = st_scr[...]
            mu = st[0:1, :] * inv_n
            var = st[1:2, :] * inv_n - mu * mu
            hn = (h_scr[pl.ds(j * _BR, _BR), :] - mu) * lax.rsqrt(var + 1e-5)
            hn = hn * g_ref[...] + be_ref[...]
            hn = jnp.maximum(hn, 0.0)
            y = jnp.dot(hn, w2_ref[...], preferred_element_type=jnp.float32)
            y = y + b2_ref[...]
            y = jnp.where(y > 0, y, 0.1 * (jnp.exp(y) - 1.0))
            bc = bc_ref[0]
            br = br_ref[0]
            ohT = (lax.broadcasted_iota(jnp.int32, (G, 1), 0) == br
                   ).astype(jnp.float32)
            sum_scr[...] += jnp.dot(ohT, y,
                                    preferred_element_type=jnp.float32)
            cnt_scr[...] += jnp.sum(ohT, axis=1, keepdims=True)
            for g in range(G):
                mg = jnp.where(bc == g, y, -1e30)
                max_scr[g:g + 1, :] = jnp.maximum(
                    max_scr[g:g + 1, :], jnp.max(mg, axis=0, keepdims=True))

            @pl.when(j == NB - 1)
            def _final():
                cnt = cnt_scr[...]
                x1 = sum_scr[...] / jnp.maximum(cnt, 1.0)
                x2 = jnp.where(cnt > 0.0, max_scr[...], 0.0)
                z = (jnp.dot(x1, c1a_ref[...],
                             preferred_element_type=jnp.float32)
                     + jnp.dot(x2, c1b_ref[...],
                               preferred_element_type=jnp.float32)
                     + jnp.dot(g0_ref[...], c1c_ref[...],
                               preferred_element_type=jnp.float32)
                     + cb1_ref[...])
                z = jnp.where(z > 0, z, 0.1 * (jnp.exp(z) - 1.0))
                lg = jnp.dot(z, cw2_ref[...],
                             preferred_element_type=jnp.float32)
                lg = lg + cb2_ref[...]
                m = jnp.max(lg, axis=1, keepdims=True)
                e = jnp.exp(lg - m)
                o_ref[...] = e / jnp.sum(e, axis=1, keepdims=True)

    return pl.pallas_call(
        body,
        grid=(2, NB),
        in_specs=[
            pl.BlockSpec((2, _BR, HH),
                         lambda i, j: (0, jnp.where(i == 0, j, NB - 1), 0)),
            pl.BlockSpec((HH, H), lambda i, j: (0, 0)),
            pl.BlockSpec((HH, H), lambda i, j: (0, 0)),
            pl.BlockSpec((1, H), lambda i, j: (0, 0)),
            pl.BlockSpec((1, H), lambda i, j: (0, 0)),
            pl.BlockSpec((1, H), lambda i, j: (0, 0)),
            pl.BlockSpec((H, H), lambda i, j: (0, 0)),
            pl.BlockSpec((1, H), lambda i, j: (0, 0)),
            pl.BlockSpec((1, _BR, 1),
                         lambda i, j: (jnp.where(i == 1, j, 0), 0, 0)),
            pl.BlockSpec((1, 1, _BR),
                         lambda i, j: (jnp.where(i == 1, j, 0), 0, 0)),
            pl.BlockSpec((G, NGF), lambda i, j: (0, 0)),
            pl.BlockSpec((H, H), lambda i, j: (0, 0)),
            pl.BlockSpec((H, H), lambda i, j: (0, 0)),
            pl.BlockSpec((NGF, H), lambda i, j: (0, 0)),
            pl.BlockSpec((1, H), lambda i, j: (0, 0)),
            pl.BlockSpec((H, NCLS), lambda i, j: (0, 0)),
            pl.BlockSpec((1, NCLS), lambda i, j: (0, 0)),
        ],
        out_specs=pl.BlockSpec((G, NCLS), lambda i, j: (0, 0)),
        out_shape=jax.ShapeDtypeStruct((G, NCLS), jnp.float32),
        scratch_shapes=[
            pltpu.VMEM((N, H), jnp.float32),
            pltpu.VMEM((8, H), jnp.float32),
            pltpu.VMEM((G, H), jnp.float32),
            pltpu.VMEM((G, H), jnp.float32),
            pltpu.VMEM((G, 1), jnp.float32),
        ],
    )(aggs, w1a, w1b, b1.reshape(1, H), gamma.reshape(1, H),
      beta.reshape(1, H), w2, b2.reshape(1, H), b_col, b_row, g0,
      c1a, c1b, c1c, cb1.reshape(1, H), cw2, cb2.reshape(1, NCLS))


def kernel(h0, coord0, g0, edge_index, batch,
           gin0_W1, gin0_b1, gin0_gamma, gin0_beta, gin0_W2, gin0_b2,
           gin1_W1, gin1_b1, gin1_gamma, gin1_beta, gin1_W2, gin1_b2,
           gin2_W1, gin2_b1, gin2_gamma, gin2_beta, gin2_W2, gin2_b2,
           clf_W1, clf_b1, clf_W2, clf_b2):
    x = jnp.concatenate([h0, coord0], axis=1)
    N, H = x.shape
    HH = H // 2
    E = edge_index.shape[1]
    src3 = edge_index[0].reshape(_NS, E // (_NS * _CH), _CH)
    dst3 = edge_index[1].reshape(_NS, E // (_NS * _CH), _CH)
    xs = jnp.stack([x[:, :HH], x[:, HH:]])
    agg_fn = _agg_build(N, H, E)
    params = [
        (gin0_W1, gin0_b1, gin0_gamma, gin0_beta, gin0_W2, gin0_b2),
        (gin1_W1, gin1_b1, gin1_gamma, gin1_beta, gin1_W2, gin1_b2),
        (gin2_W1, gin2_b1, gin2_gamma, gin2_beta, gin2_W2, gin2_b2),
    ]
    w1a = clf_W1[:H]
    w1b = clf_W1[H:2 * H]
    w1c = clf_W1[2 * H:]
    for li, (w1, b1, gamma, beta, w2, b2) in enumerate(params):
        aggs = agg_fn(xs, src3, dst3)
        if li < len(params) - 1:
            xs = _mlp_fused(aggs, w1[:HH], w1[HH:], b1, gamma, beta, w2, b2,
                            split=True)
        else:
            return _mlp_pool_clf(aggs, w1[:HH], w1[HH:], b1, gamma, beta,
                                 w2, b2, batch, g0, w1a, w1b, w1c, clf_b1,
                                 clf_W2, clf_b2)
